# Initial kernel scaffold; baseline (speedup 1.0000x reference)
import functools

import jax
import jax.numpy as jnp
import numpy as np
from jax import lax
from jax.experimental import pallas as pl
from jax.experimental.pallas import tpu as pltpu
from jax.experimental.pallas import tpu_sc as plsc

_B, _H, _W = 64, 512, 512
_HW = _H * _W
_SENSORS = 4096
_NTILES = 32
_REGION = _HW // _NTILES
_CHUNK = 128
_NCHUNK = 2
_P = _CHUNK * _NCHUNK


@functools.lru_cache(maxsize=None)
def _sensor_constants():
    dim_inds = np.asarray(
        jax.random.permutation(jax.random.key(42), _HW)
    )[:_SENSORS].astype(np.int64)

    d0 = (dim_inds // _W).astype(np.int32)
    d1 = (dim_inds % _W).astype(np.int32)
    idx_out = np.stack(
        [
            np.repeat(np.arange(_B, dtype=np.int32), _SENSORS),
            np.zeros(_B * _SENSORS, np.int32),
            np.tile(d0, _B),
            np.tile(d1, _B),
        ],
        axis=1,
    )

    gidx = np.zeros((_NTILES, _NCHUNK, _CHUNK), np.int32)
    loc = np.zeros((_NTILES, _NCHUNK, _CHUNK), np.int32)
    for t in range(_NTILES):
        mine = dim_inds[dim_inds // _REGION == t]
        c = len(mine)
        assert 0 < c <= _P
        g = np.full(_P, t * _REGION, np.int64)
        l = _REGION + (np.arange(_P) % 16)
        g[:c] = mine
        l[:c] = mine - t * _REGION
        gidx[t] = g.reshape(_NCHUNK, _CHUNK)
        loc[t] = l.reshape(_NCHUNK, _CHUNK)
    return idx_out, gidx, loc


def _sc_body(data_hbm, gidx_hbm, loc_hbm, zblk_hbm, out_hbm,
             gidx_v, idx_v, loc_v, vals_v, block_v, sem):
    wid = lax.axis_index("s") * 2 + lax.axis_index("c")
    pltpu.sync_copy(gidx_hbm.at[wid], gidx_v)
    pltpu.sync_copy(loc_hbm.at[wid], loc_v)
    pltpu.sync_copy(zblk_hbm, block_v)

    zero16 = jnp.zeros((16,), jnp.float32)

    def batch_body(b, carry):
        off = b * _HW
        for j in range(_NCHUNK):
            for k in range(_CHUNK // 16):
                sl = pl.ds(k * 16, 16)
                idx_v[j, sl] = gidx_v[j, sl] + off
        for j in range(_NCHUNK):
            pltpu.async_copy(data_hbm.at[idx_v.at[j]], vals_v.at[j], sem).wait()
        for j in range(_NCHUNK):
            for k in range(_CHUNK // 16):
                sl = pl.ds(k * 16, 16)
                plsc.store_scatter(block_v, [loc_v[j, sl]], vals_v[j, sl])
        base = off + wid * _REGION
        pltpu.sync_copy(block_v.at[pl.ds(0, _REGION)],
                        out_hbm.at[pl.ds(base, _REGION)])
        for j in range(_NCHUNK):
            for k in range(_CHUNK // 16):
                sl = pl.ds(k * 16, 16)
                plsc.store_scatter(block_v, [loc_v[j, sl]], zero16)
        return carry

    lax.fori_loop(0, _B, batch_body, 0)


@functools.lru_cache(maxsize=None)
def _sc_call():
    mesh = plsc.VectorSubcoreMesh(core_axis_name="c", subcore_axis_name="s")
    return pl.kernel(
        _sc_body,
        mesh=mesh,
        out_type=jax.ShapeDtypeStruct((_B * _HW,), jnp.float32),
        scratch_types=[
            pltpu.VMEM((_NCHUNK, _CHUNK), jnp.int32),
            pltpu.VMEM((_NCHUNK, _CHUNK), jnp.int32),
            pltpu.VMEM((_NCHUNK, _CHUNK), jnp.int32),
            pltpu.VMEM((_NCHUNK, _CHUNK), jnp.float32),
            pltpu.VMEM((_REGION + 16,), jnp.float32),
            pltpu.SemaphoreType.DMA,
        ],
    )


def kernel(data_batch):
    idx_out, gidx, loc = _sensor_constants()
    data_flat = data_batch.reshape(_B * _HW)
    values_flat = _sc_call()(
        data_flat,
        jnp.asarray(gidx),
        jnp.asarray(loc),
        jnp.zeros((_REGION + 16,), jnp.float32),
    )
    values = values_flat.reshape(_B, 1, _H, _W)
    return (values, jnp.asarray(idx_out))


# speedup vs baseline: 5.6354x; 5.6354x over previous
import functools

import jax
import jax.numpy as jnp
import numpy as np
from jax import lax
from jax.experimental import pallas as pl
from jax.experimental.pallas import tpu as pltpu
from jax.experimental.pallas import tpu_sc as plsc

_B, _H, _W = 64, 512, 512
_HW = _H * _W
_SENSORS = 4096
_NTILES = 32
_REGION = _HW // _NTILES
_CHUNK = 128
_NCHUNK = 2
_P = _CHUNK * _NCHUNK


def _threefry2x32(k1, k2, x0, x1):
    def rotl(x, d):
        return (x << np.uint32(d)) | (x >> np.uint32(32 - d))

    ks = [k1, k2, k1 ^ k2 ^ np.uint32(0x1BD11BDA)]
    rots = [(13, 15, 26, 6), (17, 29, 16, 24)]
    x = [x0 + ks[0], x1 + ks[1]]
    for i in range(5):
        for r in rots[i % 2]:
            x[0] = x[0] + x[1]
            x[1] = rotl(x[1], r) ^ x[0]
        x[0] = x[0] + ks[(i + 1) % 3]
        x[1] = x[1] + ks[(i + 2) % 3] + np.uint32(i + 1)
    return x[0], x[1]


def _np_permutation(seed, n):
    k = (np.uint32(np.int64(seed) >> 32), np.uint32(np.int64(seed) & 0xFFFFFFFF))
    x = np.arange(n, dtype=np.int32)
    num_rounds = int(np.ceil(3 * np.log(max(1, n)) / np.log(2**32 - 1)))
    for _ in range(num_rounds):
        b1, b2 = _threefry2x32(k[0], k[1], np.zeros(2, np.uint32),
                               np.arange(2, dtype=np.uint32))
        k, sub = (b1[0], b2[0]), (b1[1], b2[1])
        s1, s2 = _threefry2x32(sub[0], sub[1], np.zeros(n, np.uint32),
                               np.arange(n, dtype=np.uint32))
        x = x[np.argsort(s1 ^ s2, kind="stable")]
    return x


@functools.lru_cache(maxsize=None)
def _sensor_constants():
    dim_inds = _np_permutation(42, _HW)[:_SENSORS].astype(np.int64)

    d0 = (dim_inds // _W).astype(np.int32)
    d1 = (dim_inds % _W).astype(np.int32)
    idx_out = np.stack(
        [
            np.repeat(np.arange(_B, dtype=np.int32), _SENSORS),
            np.zeros(_B * _SENSORS, np.int32),
            np.tile(d0, _B),
            np.tile(d1, _B),
        ],
        axis=1,
    )

    gidx = np.zeros((_NTILES, _NCHUNK, _CHUNK), np.int32)
    loc = np.zeros((_NTILES, _NCHUNK, _CHUNK), np.int32)
    for t in range(_NTILES):
        mine = dim_inds[dim_inds // _REGION == t]
        c = len(mine)
        assert 0 < c <= _P
        g = np.full(_P, t * _REGION, np.int64)
        l = _REGION + (np.arange(_P) % 16)
        g[:c] = mine
        l[:c] = mine - t * _REGION
        gidx[t] = g.reshape(_NCHUNK, _CHUNK)
        loc[t] = l.reshape(_NCHUNK, _CHUNK)
    return idx_out, gidx, loc


def _sc_body(data_hbm, gidx_hbm, loc_hbm, zblk_hbm, out_hbm,
             gidx_v, idx_v, loc_v, vals_v, block_v, sem):
    wid = lax.axis_index("s") * 2 + lax.axis_index("c")
    pltpu.sync_copy(gidx_hbm.at[wid], gidx_v)
    pltpu.sync_copy(loc_hbm.at[wid], loc_v)
    pltpu.sync_copy(zblk_hbm, block_v)

    zero16 = jnp.zeros((16,), jnp.float32)

    def batch_body(b, carry):
        off = b * _HW
        for j in range(_NCHUNK):
            for k in range(_CHUNK // 16):
                sl = pl.ds(k * 16, 16)
                idx_v[j, sl] = gidx_v[j, sl] + off
        for j in range(_NCHUNK):
            pltpu.async_copy(data_hbm.at[idx_v.at[j]], vals_v.at[j], sem).wait()
        for j in range(_NCHUNK):
            for k in range(_CHUNK // 16):
                sl = pl.ds(k * 16, 16)
                plsc.store_scatter(block_v, [loc_v[j, sl]], vals_v[j, sl])
        base = off + wid * _REGION
        pltpu.sync_copy(block_v.at[pl.ds(0, _REGION)],
                        out_hbm.at[pl.ds(base, _REGION)])
        for j in range(_NCHUNK):
            for k in range(_CHUNK // 16):
                sl = pl.ds(k * 16, 16)
                plsc.store_scatter(block_v, [loc_v[j, sl]], zero16)
        return carry

    lax.fori_loop(0, _B, batch_body, 0)


@functools.lru_cache(maxsize=None)
def _sc_call():
    mesh = plsc.VectorSubcoreMesh(core_axis_name="c", subcore_axis_name="s")
    return pl.kernel(
        _sc_body,
        mesh=mesh,
        compiler_params=pltpu.CompilerParams(needs_layout_passes=False),
        out_type=jax.ShapeDtypeStruct((_B * _HW,), jnp.float32),
        scratch_types=[
            pltpu.VMEM((_NCHUNK, _CHUNK), jnp.int32),
            pltpu.VMEM((_NCHUNK, _CHUNK), jnp.int32),
            pltpu.VMEM((_NCHUNK, _CHUNK), jnp.int32),
            pltpu.VMEM((_NCHUNK, _CHUNK), jnp.float32),
            pltpu.VMEM((_REGION + 16,), jnp.float32),
            pltpu.SemaphoreType.DMA,
        ],
    )


def kernel(data_batch):
    idx_out, gidx, loc = _sensor_constants()
    data_flat = data_batch.reshape(_B * _HW)
    values_flat = _sc_call()(
        data_flat,
        jnp.asarray(gidx),
        jnp.asarray(loc),
        jnp.zeros((_REGION + 16,), jnp.float32),
    )
    values = values_flat.reshape(_B, 1, _H, _W)
    return (values, jnp.asarray(idx_out))


# speedup vs baseline: 8.1632x; 1.4485x over previous
import functools

import jax
import jax.numpy as jnp
import numpy as np
from jax import lax
from jax.experimental import pallas as pl
from jax.experimental.pallas import tpu as pltpu
from jax.experimental.pallas import tpu_sc as plsc

_B, _H, _W = 64, 512, 512
_HW = _H * _W
_SENSORS = 4096
_NTILES = 32
_REGION = _HW // _NTILES
_CHUNK = 128
_NCHUNK = 2
_P = _CHUNK * _NCHUNK


def _threefry2x32(k1, k2, x0, x1):
    def rotl(x, d):
        return (x << np.uint32(d)) | (x >> np.uint32(32 - d))

    ks = [k1, k2, k1 ^ k2 ^ np.uint32(0x1BD11BDA)]
    rots = [(13, 15, 26, 6), (17, 29, 16, 24)]
    x = [x0 + ks[0], x1 + ks[1]]
    for i in range(5):
        for r in rots[i % 2]:
            x[0] = x[0] + x[1]
            x[1] = rotl(x[1], r) ^ x[0]
        x[0] = x[0] + ks[(i + 1) % 3]
        x[1] = x[1] + ks[(i + 2) % 3] + np.uint32(i + 1)
    return x[0], x[1]


def _np_permutation(seed, n):
    k = (np.uint32(np.int64(seed) >> 32), np.uint32(np.int64(seed) & 0xFFFFFFFF))
    x = np.arange(n, dtype=np.int32)
    num_rounds = int(np.ceil(3 * np.log(max(1, n)) / np.log(2**32 - 1)))
    for _ in range(num_rounds):
        b1, b2 = _threefry2x32(k[0], k[1], np.zeros(2, np.uint32),
                               np.arange(2, dtype=np.uint32))
        k, sub = (b1[0], b2[0]), (b1[1], b2[1])
        s1, s2 = _threefry2x32(sub[0], sub[1], np.zeros(n, np.uint32),
                               np.arange(n, dtype=np.uint32))
        x = x[np.argsort(s1 ^ s2, kind="stable")]
    return x


@functools.lru_cache(maxsize=None)
def _sensor_constants():
    dim_inds = _np_permutation(42, _HW)[:_SENSORS].astype(np.int64)

    d0 = (dim_inds // _W).astype(np.int32)
    d1 = (dim_inds % _W).astype(np.int32)
    idx_out = np.stack(
        [
            np.repeat(np.arange(_B, dtype=np.int32), _SENSORS),
            np.zeros(_B * _SENSORS, np.int32),
            np.tile(d0, _B),
            np.tile(d1, _B),
        ],
        axis=1,
    )

    gidx = np.zeros((_NTILES, _NCHUNK, _CHUNK), np.int32)
    loc = np.zeros((_NTILES, _NCHUNK, _CHUNK), np.int32)
    for t in range(_NTILES):
        mine = dim_inds[dim_inds // _REGION == t]
        c = len(mine)
        assert 0 < c <= _P
        g = np.full(_P, t * _REGION, np.int64)
        l = _REGION + (np.arange(_P) % 16)
        g[:c] = mine
        l[:c] = mine - t * _REGION
        gidx[t] = g.reshape(_NCHUNK, _CHUNK)
        loc[t] = l.reshape(_NCHUNK, _CHUNK)
    return idx_out, gidx, loc


def _sc_body(data_hbm, gidx_hbm, loc_hbm, zblk_hbm, out_hbm,
             gidx_v, loc_v, idx0_v, idx1_v, vals0_v, vals1_v,
             blk0_v, blk1_v, gsem0, gsem1, wsem0, wsem1):
    wid = lax.axis_index("s") * 2 + lax.axis_index("c")
    pltpu.sync_copy(gidx_hbm.at[wid], gidx_v)
    pltpu.sync_copy(loc_hbm.at[wid], loc_v)
    pltpu.sync_copy(zblk_hbm, blk0_v)
    pltpu.sync_copy(zblk_hbm, blk1_v)

    zero16 = jnp.zeros((16,), jnp.float32)
    bufs = ((idx0_v, vals0_v, blk0_v, gsem0, wsem0),
            (idx1_v, vals1_v, blk1_v, gsem1, wsem1))

    for half, (idx_v, vals_v, _, gsem, _) in enumerate(bufs):
        for j in range(_NCHUNK):
            for k in range(_CHUNK // 16):
                sl = pl.ds(k * 16, 16)
                idx_v[j, sl] = gidx_v[j, sl] + half * _HW
        for j in range(_NCHUNK):
            pltpu.async_copy(data_hbm.at[idx_v.at[j]], vals_v.at[j], gsem)

    def pair_body(i, carry):
        for half, (idx_v, vals_v, blk_v, gsem, wsem) in enumerate(bufs):
            b = 2 * i + half
            base = b * _HW + wid * _REGION

            @pl.when(i >= 1)
            def _():
                pltpu.make_async_copy(blk_v.at[pl.ds(0, _REGION)],
                                      out_hbm.at[pl.ds(base, _REGION)],
                                      wsem).wait()
                for j in range(_NCHUNK):
                    for k in range(_CHUNK // 16):
                        sl = pl.ds(k * 16, 16)
                        plsc.store_scatter(blk_v, [loc_v[j, sl]], zero16)

            for j in range(_NCHUNK):
                pltpu.make_async_copy(data_hbm.at[idx_v.at[j]],
                                      vals_v.at[j], gsem).wait()
            for j in range(_NCHUNK):
                for k in range(_CHUNK // 16):
                    sl = pl.ds(k * 16, 16)
                    plsc.store_scatter(blk_v, [loc_v[j, sl]], vals_v[j, sl])

            pltpu.async_copy(blk_v.at[pl.ds(0, _REGION)],
                             out_hbm.at[pl.ds(base, _REGION)], wsem)

            @pl.when(i <= (_B // 2 - 2))
            def _():
                for j in range(_NCHUNK):
                    for k in range(_CHUNK // 16):
                        sl = pl.ds(k * 16, 16)
                        idx_v[j, sl] = idx_v[j, sl] + 2 * _HW
                for j in range(_NCHUNK):
                    pltpu.async_copy(data_hbm.at[idx_v.at[j]],
                                     vals_v.at[j], gsem)
        return carry

    lax.fori_loop(0, _B // 2, pair_body, 0)

    for half, (_, _, blk_v, _, wsem) in enumerate(bufs):
        b = _B - 2 + half
        pltpu.make_async_copy(
            blk_v.at[pl.ds(0, _REGION)],
            out_hbm.at[pl.ds(b * _HW + wid * _REGION, _REGION)], wsem).wait()


@functools.lru_cache(maxsize=None)
def _sc_call():
    mesh = plsc.VectorSubcoreMesh(core_axis_name="c", subcore_axis_name="s")
    return pl.kernel(
        _sc_body,
        mesh=mesh,
        compiler_params=pltpu.CompilerParams(needs_layout_passes=False),
        out_type=jax.ShapeDtypeStruct((_B * _HW,), jnp.float32),
        scratch_types=[
            pltpu.VMEM((_NCHUNK, _CHUNK), jnp.int32),
            pltpu.VMEM((_NCHUNK, _CHUNK), jnp.int32),
            pltpu.VMEM((_NCHUNK, _CHUNK), jnp.int32),
            pltpu.VMEM((_NCHUNK, _CHUNK), jnp.int32),
            pltpu.VMEM((_NCHUNK, _CHUNK), jnp.float32),
            pltpu.VMEM((_NCHUNK, _CHUNK), jnp.float32),
            pltpu.VMEM((_REGION + 16,), jnp.float32),
            pltpu.VMEM((_REGION + 16,), jnp.float32),
            pltpu.SemaphoreType.DMA,
            pltpu.SemaphoreType.DMA,
            pltpu.SemaphoreType.DMA,
            pltpu.SemaphoreType.DMA,
        ],
    )


def kernel(data_batch):
    idx_out, gidx, loc = _sensor_constants()
    data_flat = data_batch.reshape(_B * _HW)
    values_flat = _sc_call()(
        data_flat,
        jnp.asarray(gidx),
        jnp.asarray(loc),
        jnp.zeros((_REGION + 16,), jnp.float32),
    )
    values = values_flat.reshape(_B, 1, _H, _W)
    return (values, jnp.asarray(idx_out))


# speedup vs baseline: 8.7093x; 1.0669x over previous
import functools

import jax
import jax.numpy as jnp
import numpy as np
from jax import lax
from jax.experimental import pallas as pl
from jax.experimental.pallas import tpu as pltpu
from jax.experimental.pallas import tpu_sc as plsc

_B, _H, _W = 64, 512, 512
_HW = _H * _W
_SENSORS = 4096
_NTILES = 32
_REGION = _HW // _NTILES
_CHUNK = 128
_NCHUNK = 2
_P = _CHUNK * _NCHUNK


def _threefry2x32(k1, k2, x0, x1):
    def rotl(x, d):
        return (x << np.uint32(d)) | (x >> np.uint32(32 - d))

    ks = [k1, k2, k1 ^ k2 ^ np.uint32(0x1BD11BDA)]
    rots = [(13, 15, 26, 6), (17, 29, 16, 24)]
    x = [x0 + ks[0], x1 + ks[1]]
    for i in range(5):
        for r in rots[i % 2]:
            x[0] = x[0] + x[1]
            x[1] = rotl(x[1], r) ^ x[0]
        x[0] = x[0] + ks[(i + 1) % 3]
        x[1] = x[1] + ks[(i + 2) % 3] + np.uint32(i + 1)
    return x[0], x[1]


def _np_permutation(seed, n):
    k = (np.uint32(np.int64(seed) >> 32), np.uint32(np.int64(seed) & 0xFFFFFFFF))
    x = np.arange(n, dtype=np.int32)
    num_rounds = int(np.ceil(3 * np.log(max(1, n)) / np.log(2**32 - 1)))
    for _ in range(num_rounds):
        b1, b2 = _threefry2x32(k[0], k[1], np.zeros(2, np.uint32),
                               np.arange(2, dtype=np.uint32))
        k, sub = (b1[0], b2[0]), (b1[1], b2[1])
        s1, s2 = _threefry2x32(sub[0], sub[1], np.zeros(n, np.uint32),
                               np.arange(n, dtype=np.uint32))
        x = x[np.argsort(s1 ^ s2, kind="stable")]
    return x


@functools.lru_cache(maxsize=None)
def _sensor_constants():
    dim_inds = _np_permutation(42, _HW)[:_SENSORS].astype(np.int64)

    d0 = (dim_inds // _W).astype(np.int32)
    d1 = (dim_inds % _W).astype(np.int32)
    idx_out = np.stack(
        [
            np.repeat(np.arange(_B, dtype=np.int32), _SENSORS),
            np.zeros(_B * _SENSORS, np.int32),
            np.tile(d0, _B),
            np.tile(d1, _B),
        ],
        axis=1,
    )

    gidx = np.zeros((_NTILES, _NCHUNK, _CHUNK), np.int32)
    loc = np.zeros((_NTILES, _NCHUNK, _CHUNK), np.int32)
    for t in range(_NTILES):
        mine = dim_inds[dim_inds // _REGION == t]
        c = len(mine)
        assert 0 < c <= _P
        g = np.full(_P, t * _REGION, np.int64)
        l = _REGION + (np.arange(_P) % 16)
        g[:c] = mine
        l[:c] = mine - t * _REGION
        gidx[t] = g.reshape(_NCHUNK, _CHUNK)
        loc[t] = l.reshape(_NCHUNK, _CHUNK)
    return idx_out, gidx, loc


_G = 4
_NSTEP = _B // _G


def _sc_body(data_hbm, gidx_hbm, loc_hbm, zblk_hbm, out_hbm,
             gidx_v, loc_v, idx0_v, idx1_v, vals0_v, vals1_v,
             blk0_v, blk1_v, gsem0, gsem1, wsem0, wsem1):
    wid = lax.axis_index("s") * 2 + lax.axis_index("c")
    pltpu.sync_copy(gidx_hbm.at[wid], gidx_v)
    pltpu.sync_copy(loc_hbm.at[wid], loc_v)
    for blk_v in (blk0_v, blk1_v):
        for g in range(_G):
            pltpu.sync_copy(zblk_hbm, blk_v.at[g])

    zero16 = jnp.zeros((16,), jnp.float32)
    rows = [jnp.full((16,), g, jnp.int32) for g in range(_G)]
    bufs = ((idx0_v, vals0_v, blk0_v, gsem0, wsem0),
            (idx1_v, vals1_v, blk1_v, gsem1, wsem1))

    def _write_slices(blk_v, b0):
        src = blk_v.at[:, pl.ds(0, _REGION)]
        dst = out_hbm.at[pl.ds(b0, _G), pl.ds(wid * _REGION, _REGION)]
        return src, dst

    for half, (idx_v, vals_v, _, gsem, _) in enumerate(bufs):
        for g in range(_G):
            for j in range(_NCHUNK):
                r = g * _NCHUNK + j
                for k in range(_CHUNK // 16):
                    sl = pl.ds(k * 16, 16)
                    idx_v[r, sl] = gidx_v[j, sl] + (half * _G + g) * _HW
        for g in range(_G):
            for j in range(_NCHUNK):
                r = g * _NCHUNK + j
                pltpu.async_copy(data_hbm.at[idx_v.at[r]], vals_v.at[r], gsem)

    def pair_body(m, carry):
        for half, (idx_v, vals_v, blk_v, gsem, wsem) in enumerate(bufs):
            b0 = (2 * m + half) * _G

            @pl.when(m >= 1)
            def _():
                src, dst = _write_slices(blk_v, b0)
                pltpu.make_async_copy(src, dst, wsem).wait()
                for g in range(_G):
                    for j in range(_NCHUNK):
                        for k in range(_CHUNK // 16):
                            sl = pl.ds(k * 16, 16)
                            plsc.store_scatter(
                                blk_v, [rows[g], loc_v[j, sl]], zero16)

            for g in range(_G):
                for j in range(_NCHUNK):
                    r = g * _NCHUNK + j
                    pltpu.make_async_copy(data_hbm.at[idx_v.at[r]],
                                          vals_v.at[r], gsem).wait()
            for g in range(_G):
                for j in range(_NCHUNK):
                    r = g * _NCHUNK + j
                    for k in range(_CHUNK // 16):
                        sl = pl.ds(k * 16, 16)
                        plsc.store_scatter(
                            blk_v, [rows[g], loc_v[j, sl]], vals_v[r, sl])

            src, dst = _write_slices(blk_v, b0)
            pltpu.async_copy(src, dst, wsem)

            @pl.when(m <= (_NSTEP // 2 - 2))
            def _():
                for r in range(_G * _NCHUNK):
                    for k in range(_CHUNK // 16):
                        sl = pl.ds(k * 16, 16)
                        idx_v[r, sl] = idx_v[r, sl] + 2 * _G * _HW
                for r in range(_G * _NCHUNK):
                    pltpu.async_copy(data_hbm.at[idx_v.at[r]], vals_v.at[r],
                                     gsem)
        return carry

    lax.fori_loop(0, _NSTEP // 2, pair_body, 0)

    for half, (_, _, blk_v, _, wsem) in enumerate(bufs):
        b0 = (_NSTEP - 2 + half) * _G
        src, dst = _write_slices(blk_v, b0)
        pltpu.make_async_copy(src, dst, wsem).wait()


@functools.lru_cache(maxsize=None)
def _sc_call():
    mesh = plsc.VectorSubcoreMesh(core_axis_name="c", subcore_axis_name="s")
    return pl.kernel(
        _sc_body,
        mesh=mesh,
        compiler_params=pltpu.CompilerParams(needs_layout_passes=False),
        out_type=jax.ShapeDtypeStruct((_B, _HW), jnp.float32),
        scratch_types=[
            pltpu.VMEM((_NCHUNK, _CHUNK), jnp.int32),
            pltpu.VMEM((_NCHUNK, _CHUNK), jnp.int32),
            pltpu.VMEM((_G * _NCHUNK, _CHUNK), jnp.int32),
            pltpu.VMEM((_G * _NCHUNK, _CHUNK), jnp.int32),
            pltpu.VMEM((_G * _NCHUNK, _CHUNK), jnp.float32),
            pltpu.VMEM((_G * _NCHUNK, _CHUNK), jnp.float32),
            pltpu.VMEM((_G, _REGION + 16), jnp.float32),
            pltpu.VMEM((_G, _REGION + 16), jnp.float32),
            pltpu.SemaphoreType.DMA,
            pltpu.SemaphoreType.DMA,
            pltpu.SemaphoreType.DMA,
            pltpu.SemaphoreType.DMA,
        ],
    )


def kernel(data_batch):
    idx_out, gidx, loc = _sensor_constants()
    data_flat = data_batch.reshape(_B * _HW)
    values_2d = _sc_call()(
        data_flat,
        jnp.asarray(gidx),
        jnp.asarray(loc),
        jnp.zeros((_REGION + 16,), jnp.float32),
    )
    values = values_2d.reshape(_B, 1, _H, _W)
    return (values, jnp.asarray(idx_out))


# speedup vs baseline: 20.3951x; 2.3417x over previous
import functools

import jax
import jax.numpy as jnp
import numpy as np
from jax import lax
from jax.experimental import pallas as pl
from jax.experimental.pallas import tpu as pltpu
from jax.experimental.pallas import tpu_sc as plsc

_B, _H, _W = 64, 512, 512
_HW = _H * _W
_SENSORS = 4096
_NTILES = 32
_ROWS = _H // _NTILES
_REGION = _ROWS * _W
_CHUNK = 128
_NCHUNK = 2
_P = _CHUNK * _NCHUNK
_G = 2
_NSTEP = _B // _G


def _threefry2x32(k1, k2, x0, x1):
    def rotl(x, d):
        return (x << np.uint32(d)) | (x >> np.uint32(32 - d))

    ks = [k1, k2, k1 ^ k2 ^ np.uint32(0x1BD11BDA)]
    rots = [(13, 15, 26, 6), (17, 29, 16, 24)]
    x = [x0 + ks[0], x1 + ks[1]]
    for i in range(5):
        for r in rots[i % 2]:
            x[0] = x[0] + x[1]
            x[1] = rotl(x[1], r) ^ x[0]
        x[0] = x[0] + ks[(i + 1) % 3]
        x[1] = x[1] + ks[(i + 2) % 3] + np.uint32(i + 1)
    return x[0], x[1]


def _np_permutation(seed, n):
    k = (np.uint32(np.int64(seed) >> 32), np.uint32(np.int64(seed) & 0xFFFFFFFF))
    x = np.arange(n, dtype=np.int32)
    num_rounds = int(np.ceil(3 * np.log(max(1, n)) / np.log(2**32 - 1)))
    for _ in range(num_rounds):
        b1, b2 = _threefry2x32(k[0], k[1], np.zeros(2, np.uint32),
                               np.arange(2, dtype=np.uint32))
        k, sub = (b1[0], b2[0]), (b1[1], b2[1])
        s1, s2 = _threefry2x32(sub[0], sub[1], np.zeros(n, np.uint32),
                               np.arange(n, dtype=np.uint32))
        x = x[np.argsort(s1 ^ s2, kind="stable")]
    return x


@functools.lru_cache(maxsize=None)
def _sensor_constants():
    dim_inds = _np_permutation(42, _HW)[:_SENSORS].astype(np.int64)

    d0 = (dim_inds // _W).astype(np.int32)
    d1 = (dim_inds % _W).astype(np.int32)
    idx_out = np.stack(
        [
            np.repeat(np.arange(_B, dtype=np.int32), _SENSORS),
            np.zeros(_B * _SENSORS, np.int32),
            np.tile(d0, _B),
            np.tile(d1, _B),
        ],
        axis=1,
    )

    loc_j = np.zeros((_NTILES, _NCHUNK, _CHUNK), np.int32)
    loc_k = np.zeros((_NTILES, _NCHUNK, _CHUNK), np.int32)
    pad_j = np.zeros((_NTILES, _NCHUNK, _CHUNK), np.int32)
    pad_k = np.zeros((_NTILES, _NCHUNK, _CHUNK), np.int32)
    for t in range(_NTILES):
        mine = dim_inds[dim_inds // _REGION == t]
        c = len(mine)
        assert 0 < c <= _P
        jj = np.full(_P, _ROWS, np.int32)
        kk = (np.arange(_P) % 16).astype(np.int32)
        gj = np.zeros(_P, np.int32)
        gk = (np.arange(_P) % 16).astype(np.int32)
        jj[:c] = (mine // _W) % _ROWS
        kk[:c] = mine % _W
        gj[:c] = jj[:c]
        gk[:c] = kk[:c]
        loc_j[t] = gj.reshape(_NCHUNK, _CHUNK)
        loc_k[t] = gk.reshape(_NCHUNK, _CHUNK)
        pad_j[t] = jj.reshape(_NCHUNK, _CHUNK)
        pad_k[t] = kk.reshape(_NCHUNK, _CHUNK)
    return idx_out, loc_j, loc_k, pad_j, pad_k


def _sc_body(data_hbm, lj_hbm, lk_hbm, sj_hbm, sk_hbm, zblk_hbm, out_hbm,
             lj_v, lk_v, sj_v, sk_v,
             rg0_v, rg1_v, blk0_v, blk1_v, rsem0, rsem1, wsem0, wsem1):
    wid = lax.axis_index("s") * 2 + lax.axis_index("c")
    h0 = wid * _ROWS
    pltpu.sync_copy(lj_hbm.at[wid], lj_v)
    pltpu.sync_copy(lk_hbm.at[wid], lk_v)
    pltpu.sync_copy(sj_hbm.at[wid], sj_v)
    pltpu.sync_copy(sk_hbm.at[wid], sk_v)
    for blk_v in (blk0_v, blk1_v):
        for g in range(_G):
            pltpu.sync_copy(zblk_hbm, blk_v.at[g])

    zero16 = jnp.zeros((16,), jnp.float32)
    rows = [jnp.full((16,), g, jnp.int32) for g in range(_G)]
    bufs = ((rg0_v, blk0_v, rsem0, wsem0), (rg1_v, blk1_v, rsem1, wsem1))

    def _read(rg_v, b0, sem):
        return pltpu.make_async_copy(
            data_hbm.at[pl.ds(b0, _G), pl.ds(h0, _ROWS), :], rg_v, sem)

    def _write(blk_v, b0, sem):
        return pltpu.make_async_copy(
            blk_v.at[:, pl.ds(0, _ROWS), :],
            out_hbm.at[pl.ds(b0, _G), pl.ds(h0, _ROWS), :], sem)

    for half, (rg_v, _, rsem, _) in enumerate(bufs):
        _read(rg_v, half * _G, rsem).start()

    def pair_body(m, carry):
        for half, (rg_v, blk_v, rsem, wsem) in enumerate(bufs):
            b0 = (2 * m + half) * _G

            @pl.when(m >= 1)
            def _():
                _write(blk_v, b0, wsem).wait()
                for g in range(_G):
                    for c in range(_NCHUNK):
                        for k in range(_CHUNK // 16):
                            sl = pl.ds(k * 16, 16)
                            plsc.store_scatter(
                                blk_v, [rows[g], sj_v[c, sl], sk_v[c, sl]],
                                zero16)

            _read(rg_v, b0, rsem).wait()
            for g in range(_G):
                for c in range(_NCHUNK):
                    for k in range(_CHUNK // 16):
                        sl = pl.ds(k * 16, 16)
                        v = plsc.load_gather(
                            rg_v, [rows[g], lj_v[c, sl], lk_v[c, sl]])
                        plsc.store_scatter(
                            blk_v, [rows[g], sj_v[c, sl], sk_v[c, sl]], v)

            _write(blk_v, b0, wsem).start()

            @pl.when(m <= (_NSTEP // 2 - 2))
            def _():
                _read(rg_v, b0 + 2 * _G, rsem).start()
        return carry

    lax.fori_loop(0, _NSTEP // 2, pair_body, 0)

    for half, (_, blk_v, _, wsem) in enumerate(bufs):
        b0 = (_NSTEP - 2 + half) * _G
        _write(blk_v, b0, wsem).wait()


@functools.lru_cache(maxsize=None)
def _sc_call():
    mesh = plsc.VectorSubcoreMesh(core_axis_name="c", subcore_axis_name="s")
    return pl.kernel(
        _sc_body,
        mesh=mesh,
        compiler_params=pltpu.CompilerParams(needs_layout_passes=False),
        out_type=jax.ShapeDtypeStruct((_B, _H, _W), jnp.float32),
        scratch_types=[
            pltpu.VMEM((_NCHUNK, _CHUNK), jnp.int32),
            pltpu.VMEM((_NCHUNK, _CHUNK), jnp.int32),
            pltpu.VMEM((_NCHUNK, _CHUNK), jnp.int32),
            pltpu.VMEM((_NCHUNK, _CHUNK), jnp.int32),
            pltpu.VMEM((_G, _ROWS, _W), jnp.float32),
            pltpu.VMEM((_G, _ROWS, _W), jnp.float32),
            pltpu.VMEM((_G, _ROWS + 1, _W), jnp.float32),
            pltpu.VMEM((_G, _ROWS + 1, _W), jnp.float32),
            pltpu.SemaphoreType.DMA,
            pltpu.SemaphoreType.DMA,
            pltpu.SemaphoreType.DMA,
            pltpu.SemaphoreType.DMA,
        ],
    )


def kernel(data_batch):
    idx_out, loc_j, loc_k, pad_j, pad_k = _sensor_constants()
    data3 = data_batch.reshape(_B, _H, _W)
    values3 = _sc_call()(
        data3,
        jnp.asarray(loc_j),
        jnp.asarray(loc_k),
        jnp.asarray(pad_j),
        jnp.asarray(pad_k),
        jnp.zeros((_ROWS + 1, _W), jnp.float32),
    )
    values = values3.reshape(_B, 1, _H, _W)
    return (values, jnp.asarray(idx_out))
